# baseline (speedup 1.0000x reference)
import functools

import jax
import jax.numpy as jnp
from jax import lax
from jax.experimental import pallas as pl
from jax.experimental.pallas import tpu as pltpu
from jax.experimental.pallas import tpu_sc as plsc

_B, _N, _C = 4, 16384, 32
_S, _NS = 1024, 32
_R2 = 0.1 * 0.1
_DIN = 48
_NT = 512
_ST = 256
_BIG = 1e9


def _fps_body(xp_ref, yp_ref, zp_ref, xyz_ref, new_xyz_ref):
    xp = xp_ref[0]
    yp = yp_ref[0]
    zp = zp_ref[0]
    flat = (lax.broadcasted_iota(jnp.int32, (128, 128), 0) * 128
            + lax.broadcasted_iota(jnp.int32, (128, 128), 1)
            ).astype(jnp.float32)

    def body(i, carry):
        dists, idx = carry
        row = xyz_ref[0, pl.ds(idx, 1), :]
        new_xyz_ref[0, pl.ds(i, 1), :] = row
        bx = row[0:1, 0:1]
        by = row[0:1, 1:2]
        bz = row[0:1, 2:3]
        dx = xp - bx
        dy = yp - by
        dz = zp - bz
        d = (dx * dx + dy * dy) + dz * dz
        dists = jnp.minimum(dists, d)
        m = jnp.max(dists)
        cand = jnp.where(dists == m, flat, _BIG)
        nxt = jnp.min(cand).astype(jnp.int32)
        return dists, nxt

    dists0 = jnp.full((128, 128), 1e10, dtype=jnp.float32)
    lax.fori_loop(0, _S, body, (dists0, jnp.int32(0)))


def _fps(xp, yp, zp, xyz):
    return pl.pallas_call(
        _fps_body,
        grid=(_B,),
        in_specs=[
            pl.BlockSpec((1, 128, 128), lambda b: (b, 0, 0)),
            pl.BlockSpec((1, 128, 128), lambda b: (b, 0, 0)),
            pl.BlockSpec((1, 128, 128), lambda b: (b, 0, 0)),
            pl.BlockSpec((1, _N, 3), lambda b: (b, 0, 0)),
        ],
        out_specs=pl.BlockSpec((1, _S, 3), lambda b: (b, 0, 0)),
        out_shape=jax.ShapeDtypeStruct((_B, _S, 3), jnp.float32),
        compiler_params=pltpu.CompilerParams(
            dimension_semantics=("parallel",),
        ),
    )(xp, yp, zp, xyz)


def _bq_body(nx_ref, px_ref, py_ref, pz_ref, tri_ref, gidx_ref):
    b = pl.program_id(0)
    c = nx_ref[0]
    cx = c[:, 0:1]
    cy = c[:, 1:2]
    cz = c[:, 2:3]
    cn = (cx * cx + cy * cy) + cz * cz
    cxb = cx.astype(jnp.bfloat16).astype(jnp.float32)
    cyb = cy.astype(jnp.bfloat16).astype(jnp.float32)
    czb = cz.astype(jnp.bfloat16).astype(jnp.float32)
    tri = tri_ref[...]
    lane = lax.broadcasted_iota(jnp.int32, (32, _NT), 1).astype(jnp.float32)

    slot_lane = lax.broadcasted_iota(jnp.int32, (32, _NS), 1)

    def tile(carry):
        t, grp, run = carry
        sl = pl.ds(t * _NT, _NT)
        pxt = px_ref[0, 0:1, sl]
        pyt = py_ref[0, 0:1, sl]
        pzt = pz_ref[0, 0:1, sl]
        pnt = (pxt * pxt + pyt * pyt) + pzt * pzt
        pxb = pxt.astype(jnp.bfloat16).astype(jnp.float32)
        pyb = pyt.astype(jnp.bfloat16).astype(jnp.float32)
        pzb = pzt.astype(jnp.bfloat16).astype(jnp.float32)
        dot = (cxb * pxb + cyb * pyb) + czb * pzb
        sq = (cn + pnt) - 2.0 * dot
        mask = sq <= _R2
        mf = mask.astype(jnp.bfloat16)
        pref = lax.dot_general(
            mf, tri, (((1,), (0,)), ((), ())),
            preferred_element_type=jnp.float32)
        pref = pref + run
        run_new = pref[:, _NT - 1:_NT]
        colf = lane + (t * _NT).astype(jnp.float32)
        lo = jnp.min(run).astype(jnp.int32)
        hi = jnp.minimum(jnp.max(run_new), float(_NS)).astype(jnp.int32)

        def slot_body(k, g):
            kf = (k + 1).astype(jnp.float32)
            cand = jnp.where(pref >= kf, colf, _BIG)
            sm = jnp.min(cand, axis=1, keepdims=True)
            upd = jnp.minimum(g, jnp.broadcast_to(sm, (32, _NS)))
            return jnp.where(slot_lane == k, upd, g)

        grp = lax.fori_loop(lo, hi, slot_body, grp)
        return t + 1, grp, run_new

    def tile_cond(carry):
        t, _, run = carry
        return jnp.logical_and(t < _N // _NT, jnp.min(run) < float(_NS))

    grp0 = jnp.full((32, _NS), _BIG, dtype=jnp.float32)
    run0 = jnp.zeros((32, 1), dtype=jnp.float32)
    _, grp, _ = lax.while_loop(tile_cond, tile, (jnp.int32(0), grp0, run0))
    first = grp[:, 0:1]
    grp = jnp.where(grp >= _BIG, first, grp)
    gidx_ref[0] = grp.astype(jnp.int32) + b * _N


def _ball_query(new_xyz, px, py, pz, tri):
    return pl.pallas_call(
        _bq_body,
        grid=(_B, _S // 32),
        in_specs=[
            pl.BlockSpec((1, 32, 3), lambda b, s: (b, s, 0)),
            pl.BlockSpec((1, 1, _N), lambda b, s: (b, 0, 0)),
            pl.BlockSpec((1, 1, _N), lambda b, s: (b, 0, 0)),
            pl.BlockSpec((1, 1, _N), lambda b, s: (b, 0, 0)),
            pl.BlockSpec((_NT, _NT), lambda b, s: (0, 0)),
        ],
        out_specs=pl.BlockSpec((1, 32, _NS), lambda b, s: (b, s, 0)),
        out_shape=jax.ShapeDtypeStruct((_B, _S, _NS), jnp.int32),
        compiler_params=pltpu.CompilerParams(
            dimension_semantics=("parallel", "arbitrary"),
        ),
    )(new_xyz, px, py, pz, tri)


_NW = 32
_TOT = _B * _S * _NS
_PERW = _TOT // _NW
_CH = 1024


def _sc_gather(table, idx_flat):
    mesh = plsc.VectorSubcoreMesh(core_axis_name="c", subcore_axis_name="s")

    @functools.partial(
        pl.kernel,
        out_type=jax.ShapeDtypeStruct((_TOT, _DIN), jnp.float32),
        mesh=mesh,
        scratch_types=[
            pltpu.VMEM((_CH,), jnp.int32),
            pltpu.VMEM((_CH, _DIN), jnp.float32),
            pltpu.SemaphoreType.DMA,
        ],
        compiler_params=pltpu.CompilerParams(use_tc_tiling_on_sc=False),
    )
    def k(table_hbm, idx_hbm, out_hbm, idx_v, rows_v, sem):
        wid = lax.axis_index("s") * 2 + lax.axis_index("c")
        base = wid * _PERW

        @pl.loop(0, _PERW, step=_CH)
        def _(off):
            pltpu.sync_copy(idx_hbm.at[pl.ds(base + off, _CH)], idx_v)
            pltpu.async_copy(table_hbm.at[idx_v], rows_v, sem).wait()
            pltpu.sync_copy(rows_v, out_hbm.at[pl.ds(base + off, _CH)])

    return k(table, idx_flat)


def _mlp_body(g_ref, nx_ref, w1_ref, b1_ref, w2_ref, b2_ref, w3_ref, b3_ref,
              out_ref):
    g = g_ref[...]
    c = nx_ref[0]
    cc = jnp.broadcast_to(c.reshape(_ST, 1, 3), (_ST, _NS, 3))
    cc = cc.reshape(_ST * _NS, 3)
    sub = jnp.concatenate(
        [cc, jnp.zeros((_ST * _NS, _DIN - 3), jnp.float32)], axis=1)
    t = g - sub

    def mm(x, w, bias):
        y = lax.dot_general(x, w, (((1,), (0,)), ((), ())),
                            preferred_element_type=jnp.float32)
        return jnp.maximum(y + bias, 0.0)

    h = mm(t, w1_ref[...], b1_ref[...])
    h = mm(h, w2_ref[...], b2_ref[...])
    h = mm(h, w3_ref[...], b3_ref[...])
    pooled = jnp.max(h.reshape(_ST, _NS, 128), axis=1)
    out_ref[0] = pooled.T


def _mlp(gathered, new_xyz, w1p, b1, w2, b2, w3, b3):
    nblk = _S // _ST
    return pl.pallas_call(
        _mlp_body,
        grid=(_B, nblk),
        in_specs=[
            pl.BlockSpec((_ST * _NS, _DIN), lambda b, s: (b * nblk + s, 0)),
            pl.BlockSpec((1, _ST, 3), lambda b, s: (b, s, 0)),
            pl.BlockSpec((_DIN, 64), lambda b, s: (0, 0)),
            pl.BlockSpec((1, 64), lambda b, s: (0, 0)),
            pl.BlockSpec((64, 64), lambda b, s: (0, 0)),
            pl.BlockSpec((1, 64), lambda b, s: (0, 0)),
            pl.BlockSpec((64, 128), lambda b, s: (0, 0)),
            pl.BlockSpec((1, 128), lambda b, s: (0, 0)),
        ],
        out_specs=pl.BlockSpec((1, 128, _ST), lambda b, s: (b, 0, s)),
        out_shape=jax.ShapeDtypeStruct((_B, 128, _S), jnp.float32),
        compiler_params=pltpu.CompilerParams(
            dimension_semantics=("parallel", "arbitrary"),
        ),
    )(gathered, new_xyz, w1p, b1, w2, b2, w3, b3)


def kernel(xyz, features, W1, b1, W2, b2, W3, b3):
    x = xyz[:, :, 0]
    y = xyz[:, :, 1]
    z = xyz[:, :, 2]
    new_xyz = _fps(x.reshape(_B, 128, 128), y.reshape(_B, 128, 128),
                   z.reshape(_B, 128, 128), xyz)

    tri = (lax.broadcasted_iota(jnp.int32, (_NT, _NT), 0)
           <= lax.broadcasted_iota(jnp.int32, (_NT, _NT), 1)
           ).astype(jnp.bfloat16)
    gidx = _ball_query(new_xyz, x.reshape(_B, 1, _N), y.reshape(_B, 1, _N),
                       z.reshape(_B, 1, _N), tri)

    table = jnp.concatenate(
        [xyz, jnp.transpose(features, (0, 2, 1)),
         jnp.zeros((_B, _N, _DIN - 3 - _C), jnp.float32)], axis=2)
    table = table.reshape(_B * _N, _DIN)
    gathered = _sc_gather(table, gidx.reshape(_TOT))

    w1p = jnp.concatenate(
        [W1, jnp.zeros((_DIN - 3 - _C, 64), jnp.float32)], axis=0)
    new_features = _mlp(gathered, new_xyz, w1p, b1.reshape(1, 64),
                        W2, b2.reshape(1, 64), W3, b3.reshape(1, 128))
    return new_xyz, new_features


# speedup vs baseline: 1.8377x; 1.8377x over previous
import functools

import jax
import jax.numpy as jnp
from jax import lax
from jax.experimental import pallas as pl
from jax.experimental.pallas import tpu as pltpu
from jax.experimental.pallas import tpu_sc as plsc

_B, _N, _C = 4, 16384, 32
_S, _NS = 1024, 32
_R2 = 0.1 * 0.1
_DIN = 48
_NT = 512
_ST = 256
_BIG = 1e9


def _fps_body(xp_ref, yp_ref, zp_ref, xyz_ref, new_xyz_ref):
    xp = xp_ref[0]
    yp = yp_ref[0]
    zp = zp_ref[0]
    flat = (lax.broadcasted_iota(jnp.int32, (128, 128), 0) * 128
            + lax.broadcasted_iota(jnp.int32, (128, 128), 1)
            ).astype(jnp.float32)

    def body(i, carry):
        dists, idx = carry
        row = xyz_ref[0, pl.ds(idx, 1), :]
        new_xyz_ref[0, pl.ds(i, 1), :] = row
        bx = row[0:1, 0:1]
        by = row[0:1, 1:2]
        bz = row[0:1, 2:3]
        dx = xp - bx
        dy = yp - by
        dz = zp - bz
        d = (dx * dx + dy * dy) + dz * dz
        dists = jnp.minimum(dists, d)
        m = jnp.max(dists)
        cand = jnp.where(dists == m, flat, _BIG)
        nxt = jnp.min(cand).astype(jnp.int32)
        return dists, nxt

    dists0 = jnp.full((128, 128), 1e10, dtype=jnp.float32)
    lax.fori_loop(0, _S, body, (dists0, jnp.int32(0)))


def _fps(xp, yp, zp, xyz):
    return pl.pallas_call(
        _fps_body,
        grid=(_B,),
        in_specs=[
            pl.BlockSpec((1, 128, 128), lambda b: (b, 0, 0)),
            pl.BlockSpec((1, 128, 128), lambda b: (b, 0, 0)),
            pl.BlockSpec((1, 128, 128), lambda b: (b, 0, 0)),
            pl.BlockSpec((1, _N, 3), lambda b: (b, 0, 0)),
        ],
        out_specs=pl.BlockSpec((1, _S, 3), lambda b: (b, 0, 0)),
        out_shape=jax.ShapeDtypeStruct((_B, _S, 3), jnp.float32),
        compiler_params=pltpu.CompilerParams(
            dimension_semantics=("parallel",),
        ),
    )(xp, yp, zp, xyz)


def _bq_body(nx_ref, px_ref, py_ref, pz_ref, tri_ref, gidx_ref, grp_ref):
    b = pl.program_id(0)
    c = nx_ref[0]
    cx = c[:, 0:1]
    cy = c[:, 1:2]
    cz = c[:, 2:3]
    cn = (cx * cx + cy * cy) + cz * cz
    cxb = cx.astype(jnp.bfloat16).astype(jnp.float32)
    cyb = cy.astype(jnp.bfloat16).astype(jnp.float32)
    czb = cz.astype(jnp.bfloat16).astype(jnp.float32)
    tri = tri_ref[...]
    lane = lax.broadcasted_iota(jnp.int32, (32, _NT), 1).astype(jnp.float32)

    grp_ref[...] = jnp.full((32, _NS), _BIG, dtype=jnp.float32)

    def tile(carry):
        t, run = carry
        sl = pl.ds(t * _NT, _NT)
        pxt = px_ref[0, 0:1, sl]
        pyt = py_ref[0, 0:1, sl]
        pzt = pz_ref[0, 0:1, sl]
        pnt = (pxt * pxt + pyt * pyt) + pzt * pzt
        pxb = pxt.astype(jnp.bfloat16).astype(jnp.float32)
        pyb = pyt.astype(jnp.bfloat16).astype(jnp.float32)
        pzb = pzt.astype(jnp.bfloat16).astype(jnp.float32)
        dot = (cxb * pxb + cyb * pyb) + czb * pzb
        sq = (cn + pnt) - 2.0 * dot
        mask = sq <= _R2
        mf = mask.astype(jnp.bfloat16)
        pref = lax.dot_general(
            mf, tri, (((1,), (0,)), ((), ())),
            preferred_element_type=jnp.float32)
        pref = pref + run
        run_new = pref[:, _NT - 1:_NT]
        colf = lane + (t * _NT).astype(jnp.float32)
        lo = jnp.min(run)
        hi = jnp.max(run_new)
        for g in range(_NS // 8):

            @pl.when(jnp.logical_and(hi > float(g * 8), lo < float(g * 8 + 8)))
            def _(g=g):
                slots = []
                for k in range(g * 8, g * 8 + 8):
                    cand = jnp.where(pref >= (k + 1), colf, _BIG)
                    slots.append(jnp.min(cand, axis=1, keepdims=True))
                part = jnp.concatenate(slots, axis=1)
                cur = grp_ref[:, g * 8:g * 8 + 8]
                grp_ref[:, g * 8:g * 8 + 8] = jnp.minimum(cur, part)

        return t + 1, run_new

    def tile_cond(carry):
        t, run = carry
        return jnp.logical_and(t < _N // _NT, jnp.min(run) < float(_NS))

    run0 = jnp.zeros((32, 1), dtype=jnp.float32)
    lax.while_loop(tile_cond, tile, (jnp.int32(0), run0))
    grp = grp_ref[...]
    first = grp[:, 0:1]
    grp = jnp.where(grp >= _BIG, first, grp)
    gidx_ref[0] = grp.astype(jnp.int32) + b * _N


def _ball_query(new_xyz, px, py, pz, tri):
    return pl.pallas_call(
        _bq_body,
        grid=(_B, _S // 32),
        in_specs=[
            pl.BlockSpec((1, 32, 3), lambda b, s: (b, s, 0)),
            pl.BlockSpec((1, 1, _N), lambda b, s: (b, 0, 0)),
            pl.BlockSpec((1, 1, _N), lambda b, s: (b, 0, 0)),
            pl.BlockSpec((1, 1, _N), lambda b, s: (b, 0, 0)),
            pl.BlockSpec((_NT, _NT), lambda b, s: (0, 0)),
        ],
        out_specs=pl.BlockSpec((1, 32, _NS), lambda b, s: (b, s, 0)),
        out_shape=jax.ShapeDtypeStruct((_B, _S, _NS), jnp.int32),
        scratch_shapes=[pltpu.VMEM((32, _NS), jnp.float32)],
        compiler_params=pltpu.CompilerParams(
            dimension_semantics=("parallel", "arbitrary"),
        ),
    )(new_xyz, px, py, pz, tri)


_NW = 32
_TOT = _B * _S * _NS
_PERW = _TOT // _NW
_CH = 1024


def _sc_gather(table, idx_flat):
    mesh = plsc.VectorSubcoreMesh(core_axis_name="c", subcore_axis_name="s")

    @functools.partial(
        pl.kernel,
        out_type=jax.ShapeDtypeStruct((_TOT, _DIN), jnp.float32),
        mesh=mesh,
        scratch_types=[
            pltpu.VMEM((_CH,), jnp.int32),
            pltpu.VMEM((_CH, _DIN), jnp.float32),
            pltpu.SemaphoreType.DMA,
        ],
        compiler_params=pltpu.CompilerParams(use_tc_tiling_on_sc=False),
    )
    def k(table_hbm, idx_hbm, out_hbm, idx_v, rows_v, sem):
        wid = lax.axis_index("s") * 2 + lax.axis_index("c")
        base = wid * _PERW

        @pl.loop(0, _PERW, step=_CH)
        def _(off):
            pltpu.sync_copy(idx_hbm.at[pl.ds(base + off, _CH)], idx_v)
            pltpu.async_copy(table_hbm.at[idx_v], rows_v, sem).wait()
            pltpu.sync_copy(rows_v, out_hbm.at[pl.ds(base + off, _CH)])

    return k(table, idx_flat)


def _mlp_body(g_ref, nx_ref, w1_ref, b1_ref, w2_ref, b2_ref, w3_ref, b3_ref,
              out_ref):
    g = g_ref[...]
    c = nx_ref[0]
    cc = jnp.broadcast_to(c.reshape(_ST, 1, 3), (_ST, _NS, 3))
    cc = cc.reshape(_ST * _NS, 3)
    sub = jnp.concatenate(
        [cc, jnp.zeros((_ST * _NS, _DIN - 3), jnp.float32)], axis=1)
    t = g - sub

    def mm(x, w, bias):
        y = lax.dot_general(x, w, (((1,), (0,)), ((), ())),
                            preferred_element_type=jnp.float32)
        return jnp.maximum(y + bias, 0.0)

    h = mm(t, w1_ref[...], b1_ref[...])
    h = mm(h, w2_ref[...], b2_ref[...])
    h = mm(h, w3_ref[...], b3_ref[...])
    pooled = jnp.max(h.reshape(_ST, _NS, 128), axis=1)
    out_ref[0] = pooled.T


def _mlp(gathered, new_xyz, w1p, b1, w2, b2, w3, b3):
    nblk = _S // _ST
    return pl.pallas_call(
        _mlp_body,
        grid=(_B, nblk),
        in_specs=[
            pl.BlockSpec((_ST * _NS, _DIN), lambda b, s: (b * nblk + s, 0)),
            pl.BlockSpec((1, _ST, 3), lambda b, s: (b, s, 0)),
            pl.BlockSpec((_DIN, 64), lambda b, s: (0, 0)),
            pl.BlockSpec((1, 64), lambda b, s: (0, 0)),
            pl.BlockSpec((64, 64), lambda b, s: (0, 0)),
            pl.BlockSpec((1, 64), lambda b, s: (0, 0)),
            pl.BlockSpec((64, 128), lambda b, s: (0, 0)),
            pl.BlockSpec((1, 128), lambda b, s: (0, 0)),
        ],
        out_specs=pl.BlockSpec((1, 128, _ST), lambda b, s: (b, 0, s)),
        out_shape=jax.ShapeDtypeStruct((_B, 128, _S), jnp.float32),
        compiler_params=pltpu.CompilerParams(
            dimension_semantics=("parallel", "arbitrary"),
        ),
    )(gathered, new_xyz, w1p, b1, w2, b2, w3, b3)


def kernel(xyz, features, W1, b1, W2, b2, W3, b3):
    x = xyz[:, :, 0]
    y = xyz[:, :, 1]
    z = xyz[:, :, 2]
    new_xyz = _fps(x.reshape(_B, 128, 128), y.reshape(_B, 128, 128),
                   z.reshape(_B, 128, 128), xyz)

    tri = (lax.broadcasted_iota(jnp.int32, (_NT, _NT), 0)
           <= lax.broadcasted_iota(jnp.int32, (_NT, _NT), 1)
           ).astype(jnp.bfloat16)
    gidx = _ball_query(new_xyz, x.reshape(_B, 1, _N), y.reshape(_B, 1, _N),
                       z.reshape(_B, 1, _N), tri)

    table = jnp.concatenate(
        [xyz, jnp.transpose(features, (0, 2, 1)),
         jnp.zeros((_B, _N, _DIN - 3 - _C), jnp.float32)], axis=2)
    table = table.reshape(_B * _N, _DIN)
    gathered = _sc_gather(table, gidx.reshape(_TOT))

    w1p = jnp.concatenate(
        [W1, jnp.zeros((_DIN - 3 - _C, 64), jnp.float32)], axis=0)
    new_features = _mlp(gathered, new_xyz, w1p, b1.reshape(1, 64),
                        W2, b2.reshape(1, 64), W3, b3.reshape(1, 128))
    return new_xyz, new_features


# speedup vs baseline: 2.5557x; 1.3907x over previous
import functools

import jax
import jax.numpy as jnp
from jax import lax
from jax.experimental import pallas as pl
from jax.experimental.pallas import tpu as pltpu
from jax.experimental.pallas import tpu_sc as plsc

_B, _N, _C = 4, 16384, 32
_S, _NS = 1024, 32
_R2 = 0.1 * 0.1
_DIN = 48
_NT = 512
_ST = 256
_BIG = 1e9


def _fps_body(xp_ref, yp_ref, zp_ref, xyz_ref, new_xyz_ref):
    xp = xp_ref[0]
    yp = yp_ref[0]
    zp = zp_ref[0]
    flat = (lax.broadcasted_iota(jnp.int32, (128, 128), 0) * 128
            + lax.broadcasted_iota(jnp.int32, (128, 128), 1)
            ).astype(jnp.float32)

    def body(i, carry):
        dists, idx = carry
        row = xyz_ref[0, pl.ds(idx, 1), :]
        new_xyz_ref[0, pl.ds(i, 1), :] = row
        bx = row[0:1, 0:1]
        by = row[0:1, 1:2]
        bz = row[0:1, 2:3]
        dx = xp - bx
        dy = yp - by
        dz = zp - bz
        d = (dx * dx + dy * dy) + dz * dz
        dists = jnp.minimum(dists, d)
        m = jnp.max(dists)
        cand = jnp.where(dists == m, flat, _BIG)
        nxt = jnp.min(cand).astype(jnp.int32)
        return dists, nxt

    dists0 = jnp.full((128, 128), 1e10, dtype=jnp.float32)
    lax.fori_loop(0, _S, body, (dists0, jnp.int32(0)))


def _fps(xp, yp, zp, xyz):
    return pl.pallas_call(
        _fps_body,
        grid=(_B,),
        in_specs=[
            pl.BlockSpec((1, 128, 128), lambda b: (b, 0, 0)),
            pl.BlockSpec((1, 128, 128), lambda b: (b, 0, 0)),
            pl.BlockSpec((1, 128, 128), lambda b: (b, 0, 0)),
            pl.BlockSpec((1, _N, 3), lambda b: (b, 0, 0)),
        ],
        out_specs=pl.BlockSpec((1, _S, 3), lambda b: (b, 0, 0)),
        out_shape=jax.ShapeDtypeStruct((_B, _S, 3), jnp.float32),
        compiler_params=pltpu.CompilerParams(
            dimension_semantics=("parallel",),
        ),
    )(xp, yp, zp, xyz)


def _bq_body(nx_ref, px_ref, py_ref, pz_ref, tri_ref, gidx_ref, grp_ref,
             run_ref):
    b = pl.program_id(0)
    c = nx_ref[0]
    cx = c[:, 0:1]
    cy = c[:, 1:2]
    cz = c[:, 2:3]
    cn = (cx * cx + cy * cy) + cz * cz
    cxb = cx.astype(jnp.bfloat16).astype(jnp.float32)
    cyb = cy.astype(jnp.bfloat16).astype(jnp.float32)
    czb = cz.astype(jnp.bfloat16).astype(jnp.float32)
    tri = tri_ref[...]
    lane = lax.broadcasted_iota(jnp.int32, (32, _NT), 1).astype(jnp.float32)

    grp_ref[...] = jnp.full((32, _NS), _BIG, dtype=jnp.float32)
    run_ref[...] = jnp.zeros((32, 1), dtype=jnp.float32)

    def tile(t, _):
        run = run_ref[...]

        @pl.when(jnp.min(run) < float(_NS))
        def _():
            sl = pl.ds(t * _NT, _NT)
            pxt = px_ref[0, 0:1, sl]
            pyt = py_ref[0, 0:1, sl]
            pzt = pz_ref[0, 0:1, sl]
            pnt = (pxt * pxt + pyt * pyt) + pzt * pzt
            pxb = pxt.astype(jnp.bfloat16).astype(jnp.float32)
            pyb = pyt.astype(jnp.bfloat16).astype(jnp.float32)
            pzb = pzt.astype(jnp.bfloat16).astype(jnp.float32)
            dot = (cxb * pxb + cyb * pyb) + czb * pzb
            sq = (cn + pnt) - 2.0 * dot
            mask = sq <= _R2
            mf = mask.astype(jnp.bfloat16)
            pref = lax.dot_general(
                mf, tri, (((1,), (0,)), ((), ())),
                preferred_element_type=jnp.float32)
            pref = pref + run
            colf = lane + (t * _NT).astype(jnp.float32)
            slots = []
            for k in range(_NS):
                cand = jnp.where(pref >= (k + 1), colf, _BIG)
                slots.append(jnp.min(cand, axis=1, keepdims=True))
            tile_min = jnp.concatenate(slots, axis=1)
            grp_ref[...] = jnp.minimum(grp_ref[...], tile_min)
            run_ref[...] = pref[:, _NT - 1:_NT]

        return 0

    lax.fori_loop(0, _N // _NT, tile, 0)
    grp = grp_ref[...]
    first = grp[:, 0:1]
    grp = jnp.where(grp >= _BIG, first, grp)
    gidx_ref[0] = grp.astype(jnp.int32) + b * _N


def _ball_query(new_xyz, px, py, pz, tri):
    return pl.pallas_call(
        _bq_body,
        grid=(_B, _S // 32),
        in_specs=[
            pl.BlockSpec((1, 32, 3), lambda b, s: (b, s, 0)),
            pl.BlockSpec((1, 1, _N), lambda b, s: (b, 0, 0)),
            pl.BlockSpec((1, 1, _N), lambda b, s: (b, 0, 0)),
            pl.BlockSpec((1, 1, _N), lambda b, s: (b, 0, 0)),
            pl.BlockSpec((_NT, _NT), lambda b, s: (0, 0)),
        ],
        out_specs=pl.BlockSpec((1, 32, _NS), lambda b, s: (b, s, 0)),
        out_shape=jax.ShapeDtypeStruct((_B, _S, _NS), jnp.int32),
        scratch_shapes=[pltpu.VMEM((32, _NS), jnp.float32),
                        pltpu.VMEM((32, 1), jnp.float32)],
        compiler_params=pltpu.CompilerParams(
            dimension_semantics=("parallel", "arbitrary"),
        ),
    )(new_xyz, px, py, pz, tri)


_NW = 32
_TOT = _B * _S * _NS
_PERW = _TOT // _NW
_CH = 1024


def _sc_gather(table, idx_flat):
    mesh = plsc.VectorSubcoreMesh(core_axis_name="c", subcore_axis_name="s")

    @functools.partial(
        pl.kernel,
        out_type=jax.ShapeDtypeStruct((_TOT, _DIN), jnp.float32),
        mesh=mesh,
        scratch_types=[
            pltpu.VMEM((_CH,), jnp.int32),
            pltpu.VMEM((_CH, _DIN), jnp.float32),
            pltpu.SemaphoreType.DMA,
        ],
        compiler_params=pltpu.CompilerParams(use_tc_tiling_on_sc=False),
    )
    def k(table_hbm, idx_hbm, out_hbm, idx_v, rows_v, sem):
        wid = lax.axis_index("s") * 2 + lax.axis_index("c")
        base = wid * _PERW

        @pl.loop(0, _PERW, step=_CH)
        def _(off):
            pltpu.sync_copy(idx_hbm.at[pl.ds(base + off, _CH)], idx_v)
            pltpu.async_copy(table_hbm.at[idx_v], rows_v, sem).wait()
            pltpu.sync_copy(rows_v, out_hbm.at[pl.ds(base + off, _CH)])

    return k(table, idx_flat)


def _mlp_body(g_ref, nx_ref, w1_ref, b1_ref, w2_ref, b2_ref, w3_ref, b3_ref,
              out_ref):
    g = g_ref[...]
    c = nx_ref[0]
    cc = jnp.broadcast_to(c.reshape(_ST, 1, 3), (_ST, _NS, 3))
    cc = cc.reshape(_ST * _NS, 3)
    sub = jnp.concatenate(
        [cc, jnp.zeros((_ST * _NS, _DIN - 3), jnp.float32)], axis=1)
    t = g - sub

    def mm(x, w, bias):
        y = lax.dot_general(x, w, (((1,), (0,)), ((), ())),
                            preferred_element_type=jnp.float32)
        return jnp.maximum(y + bias, 0.0)

    h = mm(t, w1_ref[...], b1_ref[...])
    h = mm(h, w2_ref[...], b2_ref[...])
    h = mm(h, w3_ref[...], b3_ref[...])
    pooled = jnp.max(h.reshape(_ST, _NS, 128), axis=1)
    out_ref[0] = pooled.T


def _mlp(gathered, new_xyz, w1p, b1, w2, b2, w3, b3):
    nblk = _S // _ST
    return pl.pallas_call(
        _mlp_body,
        grid=(_B, nblk),
        in_specs=[
            pl.BlockSpec((_ST * _NS, _DIN), lambda b, s: (b * nblk + s, 0)),
            pl.BlockSpec((1, _ST, 3), lambda b, s: (b, s, 0)),
            pl.BlockSpec((_DIN, 64), lambda b, s: (0, 0)),
            pl.BlockSpec((1, 64), lambda b, s: (0, 0)),
            pl.BlockSpec((64, 64), lambda b, s: (0, 0)),
            pl.BlockSpec((1, 64), lambda b, s: (0, 0)),
            pl.BlockSpec((64, 128), lambda b, s: (0, 0)),
            pl.BlockSpec((1, 128), lambda b, s: (0, 0)),
        ],
        out_specs=pl.BlockSpec((1, 128, _ST), lambda b, s: (b, 0, s)),
        out_shape=jax.ShapeDtypeStruct((_B, 128, _S), jnp.float32),
        compiler_params=pltpu.CompilerParams(
            dimension_semantics=("parallel", "arbitrary"),
        ),
    )(gathered, new_xyz, w1p, b1, w2, b2, w3, b3)


def kernel(xyz, features, W1, b1, W2, b2, W3, b3):
    x = xyz[:, :, 0]
    y = xyz[:, :, 1]
    z = xyz[:, :, 2]
    new_xyz = _fps(x.reshape(_B, 128, 128), y.reshape(_B, 128, 128),
                   z.reshape(_B, 128, 128), xyz)

    tri = (lax.broadcasted_iota(jnp.int32, (_NT, _NT), 0)
           <= lax.broadcasted_iota(jnp.int32, (_NT, _NT), 1)
           ).astype(jnp.bfloat16)
    gidx = _ball_query(new_xyz, x.reshape(_B, 1, _N), y.reshape(_B, 1, _N),
                       z.reshape(_B, 1, _N), tri)

    table = jnp.concatenate(
        [xyz, jnp.transpose(features, (0, 2, 1)),
         jnp.zeros((_B, _N, _DIN - 3 - _C), jnp.float32)], axis=2)
    table = table.reshape(_B * _N, _DIN)
    gathered = _sc_gather(table, gidx.reshape(_TOT))

    w1p = jnp.concatenate(
        [W1, jnp.zeros((_DIN - 3 - _C, 64), jnp.float32)], axis=0)
    new_features = _mlp(gathered, new_xyz, w1p, b1.reshape(1, 64),
                        W2, b2.reshape(1, 64), W3, b3.reshape(1, 128))
    return new_xyz, new_features


# speedup vs baseline: 2.6931x; 1.0538x over previous
import functools

import jax
import jax.numpy as jnp
from jax import lax
from jax.experimental import pallas as pl
from jax.experimental.pallas import tpu as pltpu
from jax.experimental.pallas import tpu_sc as plsc

_B, _N, _C = 4, 16384, 32
_S, _NS = 1024, 32
_R2 = 0.1 * 0.1
_DIN = 48
_NT = 512
_ST = 256
_BIG = 1e9


def _fps_body(xp_ref, yp_ref, zp_ref, new_xyz_ref):
    xp = xp_ref[0]
    yp = yp_ref[0]
    zp = zp_ref[0]
    flat = (lax.broadcasted_iota(jnp.int32, (128, 128), 0) * 128
            + lax.broadcasted_iota(jnp.int32, (128, 128), 1)
            ).astype(jnp.float32)

    def body(i, carry):
        dists, cx, cy, cz = carry
        new_xyz_ref[0, pl.ds(i, 1), :] = jnp.concatenate(
            [cx, cy, cz], axis=1)
        dx = xp - cx
        dy = yp - cy
        dz = zp - cz
        d = (dx * dx + dy * dy) + dz * dz
        dists = jnp.minimum(dists, d)
        m = jnp.max(dists)
        cand = jnp.where(dists == m, flat, _BIG)
        m2 = jnp.min(cand)
        onehot = cand == m2
        nx = jnp.sum(jnp.where(onehot, xp, 0.0)).reshape(1, 1)
        ny = jnp.sum(jnp.where(onehot, yp, 0.0)).reshape(1, 1)
        nz = jnp.sum(jnp.where(onehot, zp, 0.0)).reshape(1, 1)
        return dists, nx, ny, nz

    dists0 = jnp.full((128, 128), 1e10, dtype=jnp.float32)
    c0 = [v[0:1, 0:1] for v in (xp, yp, zp)]
    lax.fori_loop(0, _S, body, (dists0, c0[0], c0[1], c0[2]))


def _fps(xp, yp, zp):
    return pl.pallas_call(
        _fps_body,
        grid=(_B,),
        in_specs=[
            pl.BlockSpec((1, 128, 128), lambda b: (b, 0, 0)),
            pl.BlockSpec((1, 128, 128), lambda b: (b, 0, 0)),
            pl.BlockSpec((1, 128, 128), lambda b: (b, 0, 0)),
        ],
        out_specs=pl.BlockSpec((1, _S, 3), lambda b: (b, 0, 0)),
        out_shape=jax.ShapeDtypeStruct((_B, _S, 3), jnp.float32),
        compiler_params=pltpu.CompilerParams(
            dimension_semantics=("parallel",),
        ),
    )(xp, yp, zp)


def _bq_body(nx_ref, px_ref, py_ref, pz_ref, tri_ref, gidx_ref):
    b = pl.program_id(0)
    c = nx_ref[0]
    cx = c[:, 0:1]
    cy = c[:, 1:2]
    cz = c[:, 2:3]
    cn = (cx * cx + cy * cy) + cz * cz
    cxb = cx.astype(jnp.bfloat16).astype(jnp.float32)
    cyb = cy.astype(jnp.bfloat16).astype(jnp.float32)
    czb = cz.astype(jnp.bfloat16).astype(jnp.float32)
    tri = tri_ref[...]
    lane = lax.broadcasted_iota(jnp.int32, (32, _NT), 1).astype(jnp.float32)

    def tile(t, carry):
        grp, run = carry
        sl = pl.ds(t * _NT, _NT)
        pxt = px_ref[0, 0:1, sl]
        pyt = py_ref[0, 0:1, sl]
        pzt = pz_ref[0, 0:1, sl]
        pnt = (pxt * pxt + pyt * pyt) + pzt * pzt
        pxb = pxt.astype(jnp.bfloat16).astype(jnp.float32)
        pyb = pyt.astype(jnp.bfloat16).astype(jnp.float32)
        pzb = pzt.astype(jnp.bfloat16).astype(jnp.float32)
        dot = (cxb * pxb + cyb * pyb) + czb * pzb
        sq = (cn + pnt) - 2.0 * dot
        mask = sq <= _R2
        mf = mask.astype(jnp.bfloat16)
        pref = lax.dot_general(
            mf, tri, (((1,), (0,)), ((), ())),
            preferred_element_type=jnp.float32)
        pref = pref + run
        colf = lane + (t * _NT).astype(jnp.float32)
        slots = []
        for k in range(_NS):
            cand = jnp.where(pref >= (k + 1), colf, _BIG)
            slots.append(jnp.min(cand, axis=1, keepdims=True))
        tile_min = jnp.concatenate(slots, axis=1)
        grp = jnp.minimum(grp, tile_min)
        run = pref[:, _NT - 1:_NT]
        return grp, run

    grp0 = jnp.full((32, _NS), _BIG, dtype=jnp.float32)
    run0 = jnp.zeros((32, 1), dtype=jnp.float32)
    grp, _ = lax.fori_loop(0, _N // _NT, tile, (grp0, run0))
    first = grp[:, 0:1]
    grp = jnp.where(grp >= _BIG, first, grp)
    gidx_ref[0] = grp.astype(jnp.int32) + b * _N


def _ball_query(new_xyz, px, py, pz, tri):
    return pl.pallas_call(
        _bq_body,
        grid=(_B, _S // 32),
        in_specs=[
            pl.BlockSpec((1, 32, 3), lambda b, s: (b, s, 0)),
            pl.BlockSpec((1, 1, _N), lambda b, s: (b, 0, 0)),
            pl.BlockSpec((1, 1, _N), lambda b, s: (b, 0, 0)),
            pl.BlockSpec((1, 1, _N), lambda b, s: (b, 0, 0)),
            pl.BlockSpec((_NT, _NT), lambda b, s: (0, 0)),
        ],
        out_specs=pl.BlockSpec((1, 32, _NS), lambda b, s: (b, s, 0)),
        out_shape=jax.ShapeDtypeStruct((_B, _S, _NS), jnp.int32),
        compiler_params=pltpu.CompilerParams(
            dimension_semantics=("parallel", "arbitrary"),
        ),
    )(new_xyz, px, py, pz, tri)


_NW = 32
_TOT = _B * _S * _NS
_PERW = _TOT // _NW
_CH = 1024


def _sc_gather(table, idx_flat):
    mesh = plsc.VectorSubcoreMesh(core_axis_name="c", subcore_axis_name="s")

    @functools.partial(
        pl.kernel,
        out_type=jax.ShapeDtypeStruct((_TOT, _DIN), jnp.float32),
        mesh=mesh,
        scratch_types=[
            pltpu.VMEM((_CH,), jnp.int32),
            pltpu.VMEM((_CH, _DIN), jnp.float32),
            pltpu.SemaphoreType.DMA,
        ],
        compiler_params=pltpu.CompilerParams(use_tc_tiling_on_sc=False),
    )
    def k(table_hbm, idx_hbm, out_hbm, idx_v, rows_v, sem):
        wid = lax.axis_index("s") * 2 + lax.axis_index("c")
        base = wid * _PERW

        @pl.loop(0, _PERW, step=_CH)
        def _(off):
            pltpu.sync_copy(idx_hbm.at[pl.ds(base + off, _CH)], idx_v)
            pltpu.async_copy(table_hbm.at[idx_v], rows_v, sem).wait()
            pltpu.sync_copy(rows_v, out_hbm.at[pl.ds(base + off, _CH)])

    return k(table, idx_flat)


def _mlp_body(g_ref, nx_ref, w1_ref, b1_ref, w2_ref, b2_ref, w3_ref, b3_ref,
              out_ref):
    g = g_ref[...]
    c = nx_ref[0]
    cc = jnp.broadcast_to(c.reshape(_ST, 1, 3), (_ST, _NS, 3))
    cc = cc.reshape(_ST * _NS, 3)
    sub = jnp.concatenate(
        [cc, jnp.zeros((_ST * _NS, _DIN - 3), jnp.float32)], axis=1)
    t = g - sub

    def mm(x, w, bias):
        y = lax.dot_general(x, w, (((1,), (0,)), ((), ())),
                            preferred_element_type=jnp.float32)
        return jnp.maximum(y + bias, 0.0)

    h = mm(t, w1_ref[...], b1_ref[...])
    h = mm(h, w2_ref[...], b2_ref[...])
    h = mm(h, w3_ref[...], b3_ref[...])
    pooled = jnp.max(h.reshape(_ST, _NS, 128), axis=1)
    out_ref[0] = pooled.T


def _mlp(gathered, new_xyz, w1p, b1, w2, b2, w3, b3):
    nblk = _S // _ST
    return pl.pallas_call(
        _mlp_body,
        grid=(_B, nblk),
        in_specs=[
            pl.BlockSpec((_ST * _NS, _DIN), lambda b, s: (b * nblk + s, 0)),
            pl.BlockSpec((1, _ST, 3), lambda b, s: (b, s, 0)),
            pl.BlockSpec((_DIN, 64), lambda b, s: (0, 0)),
            pl.BlockSpec((1, 64), lambda b, s: (0, 0)),
            pl.BlockSpec((64, 64), lambda b, s: (0, 0)),
            pl.BlockSpec((1, 64), lambda b, s: (0, 0)),
            pl.BlockSpec((64, 128), lambda b, s: (0, 0)),
            pl.BlockSpec((1, 128), lambda b, s: (0, 0)),
        ],
        out_specs=pl.BlockSpec((1, 128, _ST), lambda b, s: (b, 0, s)),
        out_shape=jax.ShapeDtypeStruct((_B, 128, _S), jnp.float32),
        compiler_params=pltpu.CompilerParams(
            dimension_semantics=("parallel", "arbitrary"),
        ),
    )(gathered, new_xyz, w1p, b1, w2, b2, w3, b3)


def kernel(xyz, features, W1, b1, W2, b2, W3, b3):
    x = xyz[:, :, 0]
    y = xyz[:, :, 1]
    z = xyz[:, :, 2]
    new_xyz = _fps(x.reshape(_B, 128, 128), y.reshape(_B, 128, 128),
                   z.reshape(_B, 128, 128))

    tri = (lax.broadcasted_iota(jnp.int32, (_NT, _NT), 0)
           <= lax.broadcasted_iota(jnp.int32, (_NT, _NT), 1)
           ).astype(jnp.bfloat16)
    gidx = _ball_query(new_xyz, x.reshape(_B, 1, _N), y.reshape(_B, 1, _N),
                       z.reshape(_B, 1, _N), tri)

    table = jnp.concatenate(
        [xyz, jnp.transpose(features, (0, 2, 1)),
         jnp.zeros((_B, _N, _DIN - 3 - _C), jnp.float32)], axis=2)
    table = table.reshape(_B * _N, _DIN)
    gathered = _sc_gather(table, gidx.reshape(_TOT))

    w1p = jnp.concatenate(
        [W1, jnp.zeros((_DIN - 3 - _C, 64), jnp.float32)], axis=0)
    new_features = _mlp(gathered, new_xyz, w1p, b1.reshape(1, 64),
                        W2, b2.reshape(1, 64), W3, b3.reshape(1, 128))
    return new_xyz, new_features


# speedup vs baseline: 3.3760x; 1.2536x over previous
import functools

import jax
import jax.numpy as jnp
from jax import lax
from jax.experimental import pallas as pl
from jax.experimental.pallas import tpu as pltpu
from jax.experimental.pallas import tpu_sc as plsc

_B, _N, _C = 4, 16384, 32
_S, _NS = 1024, 32
_R2 = 0.1 * 0.1
_DIN = 48
_NT = 512
_ST = 256
_BIG = 1e9


_FPP = 2


def _fps_body(xp_ref, yp_ref, zp_ref, new_xyz_ref):
    xs = [xp_ref[a] for a in range(_FPP)]
    ys = [yp_ref[a] for a in range(_FPP)]
    zs = [zp_ref[a] for a in range(_FPP)]
    flat = (lax.broadcasted_iota(jnp.int32, (128, 128), 0) * 128
            + lax.broadcasted_iota(jnp.int32, (128, 128), 1)
            ).astype(jnp.float32)

    def body(i, carry):
        out = []
        for a in range(_FPP):
            dists, cx, cy, cz = carry[a]
            new_xyz_ref[a, pl.ds(i, 1), :] = jnp.concatenate(
                [cx, cy, cz], axis=1)
            dx = xs[a] - cx
            dy = ys[a] - cy
            dz = zs[a] - cz
            d = (dx * dx + dy * dy) + dz * dz
            dists = jnp.minimum(dists, d)
            m = jnp.max(dists)
            cand = jnp.where(dists == m, flat, _BIG)
            m2 = jnp.min(cand)
            onehot = cand == m2
            nx = jnp.sum(jnp.where(onehot, xs[a], 0.0)).reshape(1, 1)
            ny = jnp.sum(jnp.where(onehot, ys[a], 0.0)).reshape(1, 1)
            nz = jnp.sum(jnp.where(onehot, zs[a], 0.0)).reshape(1, 1)
            out.append((dists, nx, ny, nz))
        return tuple(out)

    dists0 = jnp.full((128, 128), 1e10, dtype=jnp.float32)
    init = tuple(
        (dists0, xs[a][0:1, 0:1], ys[a][0:1, 0:1], zs[a][0:1, 0:1])
        for a in range(_FPP))
    lax.fori_loop(0, _S, body, init)


def _fps(xp, yp, zp):
    return pl.pallas_call(
        _fps_body,
        grid=(_B // _FPP,),
        in_specs=[
            pl.BlockSpec((_FPP, 128, 128), lambda b: (b, 0, 0)),
            pl.BlockSpec((_FPP, 128, 128), lambda b: (b, 0, 0)),
            pl.BlockSpec((_FPP, 128, 128), lambda b: (b, 0, 0)),
        ],
        out_specs=pl.BlockSpec((_FPP, _S, 3), lambda b: (b, 0, 0)),
        out_shape=jax.ShapeDtypeStruct((_B, _S, 3), jnp.float32),
        compiler_params=pltpu.CompilerParams(
            dimension_semantics=("arbitrary",),
        ),
    )(xp, yp, zp)


def _bq_body(nx_ref, px_ref, py_ref, pz_ref, tri_ref, gidx_ref):
    b = pl.program_id(0)
    c = nx_ref[0]
    cx = c[:, 0:1]
    cy = c[:, 1:2]
    cz = c[:, 2:3]
    cn = (cx * cx + cy * cy) + cz * cz
    cxb = cx.astype(jnp.bfloat16).astype(jnp.float32)
    cyb = cy.astype(jnp.bfloat16).astype(jnp.float32)
    czb = cz.astype(jnp.bfloat16).astype(jnp.float32)
    tri = tri_ref[...]
    lane = lax.broadcasted_iota(jnp.int32, (32, _NT), 1).astype(jnp.float32)

    def one_tile(t, grp, run):
        sl = pl.ds(t * _NT, _NT)
        pxt = px_ref[0, 0:1, sl]
        pyt = py_ref[0, 0:1, sl]
        pzt = pz_ref[0, 0:1, sl]
        pnt = (pxt * pxt + pyt * pyt) + pzt * pzt
        pxb = pxt.astype(jnp.bfloat16).astype(jnp.float32)
        pyb = pyt.astype(jnp.bfloat16).astype(jnp.float32)
        pzb = pzt.astype(jnp.bfloat16).astype(jnp.float32)
        dot = (cxb * pxb + cyb * pyb) + czb * pzb
        sq = (cn + pnt) - 2.0 * dot
        mask = sq <= _R2
        mf = mask.astype(jnp.bfloat16)
        pref = lax.dot_general(
            mf, tri, (((1,), (0,)), ((), ())),
            preferred_element_type=jnp.float32)
        pref = pref + run
        colf = lane + (t * _NT).astype(jnp.float32)
        slots = []
        for k in range(_NS):
            cand = jnp.where(pref >= (k + 1), colf, _BIG)
            slots.append(jnp.min(cand, axis=1, keepdims=True))
        tile_min = jnp.concatenate(slots, axis=1)
        return jnp.minimum(grp, tile_min), pref[:, _NT - 1:_NT]

    def tile8(t, carry):
        grp, run = carry
        for u in range(8):
            grp, run = one_tile(8 * t + u, grp, run)
        return grp, run

    grp0 = jnp.full((32, _NS), _BIG, dtype=jnp.float32)
    run0 = jnp.zeros((32, 1), dtype=jnp.float32)
    grp, _ = lax.fori_loop(0, _N // _NT // 8, tile8, (grp0, run0))
    first = grp[:, 0:1]
    grp = jnp.where(grp >= _BIG, first, grp)
    gidx_ref[0] = grp.astype(jnp.int32) + b * _N


def _ball_query(new_xyz, px, py, pz, tri):
    return pl.pallas_call(
        _bq_body,
        grid=(_B, _S // 32),
        in_specs=[
            pl.BlockSpec((1, 32, 3), lambda b, s: (b, s, 0)),
            pl.BlockSpec((1, 1, _N), lambda b, s: (b, 0, 0)),
            pl.BlockSpec((1, 1, _N), lambda b, s: (b, 0, 0)),
            pl.BlockSpec((1, 1, _N), lambda b, s: (b, 0, 0)),
            pl.BlockSpec((_NT, _NT), lambda b, s: (0, 0)),
        ],
        out_specs=pl.BlockSpec((1, 32, _NS), lambda b, s: (b, s, 0)),
        out_shape=jax.ShapeDtypeStruct((_B, _S, _NS), jnp.int32),
        compiler_params=pltpu.CompilerParams(
            dimension_semantics=("parallel", "arbitrary"),
        ),
    )(new_xyz, px, py, pz, tri)


_NW = 32
_TOT = _B * _S * _NS
_PERW = _TOT // _NW
_CH = 1024


def _sc_gather(table, idx_flat):
    mesh = plsc.VectorSubcoreMesh(core_axis_name="c", subcore_axis_name="s")

    @functools.partial(
        pl.kernel,
        out_type=jax.ShapeDtypeStruct((_TOT, _DIN), jnp.float32),
        mesh=mesh,
        scratch_types=[
            pltpu.VMEM((_CH,), jnp.int32),
            pltpu.VMEM((_CH, _DIN), jnp.float32),
            pltpu.SemaphoreType.DMA,
        ],
        compiler_params=pltpu.CompilerParams(use_tc_tiling_on_sc=False),
    )
    def k(table_hbm, idx_hbm, out_hbm, idx_v, rows_v, sem):
        wid = lax.axis_index("s") * 2 + lax.axis_index("c")
        base = wid * _PERW

        @pl.loop(0, _PERW, step=_CH)
        def _(off):
            pltpu.sync_copy(idx_hbm.at[pl.ds(base + off, _CH)], idx_v)
            pltpu.async_copy(table_hbm.at[idx_v], rows_v, sem).wait()
            pltpu.sync_copy(rows_v, out_hbm.at[pl.ds(base + off, _CH)])

    return k(table, idx_flat)


def _mlp_body(g_ref, nx_ref, w1_ref, b1_ref, w2_ref, b2_ref, w3_ref, b3_ref,
              out_ref):
    g = g_ref[...]
    c = nx_ref[0]
    cc = jnp.broadcast_to(c.reshape(_ST, 1, 3), (_ST, _NS, 3))
    cc = cc.reshape(_ST * _NS, 3)
    sub = jnp.concatenate(
        [cc, jnp.zeros((_ST * _NS, _DIN - 3), jnp.float32)], axis=1)
    t = g - sub

    def mm(x, w, bias):
        y = lax.dot_general(x, w, (((1,), (0,)), ((), ())),
                            preferred_element_type=jnp.float32)
        return jnp.maximum(y + bias, 0.0)

    h = mm(t, w1_ref[...], b1_ref[...])
    h = mm(h, w2_ref[...], b2_ref[...])
    h = mm(h, w3_ref[...], b3_ref[...])
    pooled = jnp.max(h.reshape(_ST, _NS, 128), axis=1)
    out_ref[0] = pooled.T


def _mlp(gathered, new_xyz, w1p, b1, w2, b2, w3, b3):
    nblk = _S // _ST
    return pl.pallas_call(
        _mlp_body,
        grid=(_B, nblk),
        in_specs=[
            pl.BlockSpec((_ST * _NS, _DIN), lambda b, s: (b * nblk + s, 0)),
            pl.BlockSpec((1, _ST, 3), lambda b, s: (b, s, 0)),
            pl.BlockSpec((_DIN, 64), lambda b, s: (0, 0)),
            pl.BlockSpec((1, 64), lambda b, s: (0, 0)),
            pl.BlockSpec((64, 64), lambda b, s: (0, 0)),
            pl.BlockSpec((1, 64), lambda b, s: (0, 0)),
            pl.BlockSpec((64, 128), lambda b, s: (0, 0)),
            pl.BlockSpec((1, 128), lambda b, s: (0, 0)),
        ],
        out_specs=pl.BlockSpec((1, 128, _ST), lambda b, s: (b, 0, s)),
        out_shape=jax.ShapeDtypeStruct((_B, 128, _S), jnp.float32),
        compiler_params=pltpu.CompilerParams(
            dimension_semantics=("parallel", "arbitrary"),
        ),
    )(gathered, new_xyz, w1p, b1, w2, b2, w3, b3)


def kernel(xyz, features, W1, b1, W2, b2, W3, b3):
    x = xyz[:, :, 0]
    y = xyz[:, :, 1]
    z = xyz[:, :, 2]
    new_xyz = _fps(x.reshape(_B, 128, 128), y.reshape(_B, 128, 128),
                   z.reshape(_B, 128, 128))

    tri = (lax.broadcasted_iota(jnp.int32, (_NT, _NT), 0)
           <= lax.broadcasted_iota(jnp.int32, (_NT, _NT), 1)
           ).astype(jnp.bfloat16)
    gidx = _ball_query(new_xyz, x.reshape(_B, 1, _N), y.reshape(_B, 1, _N),
                       z.reshape(_B, 1, _N), tri)

    table = jnp.concatenate(
        [xyz, jnp.transpose(features, (0, 2, 1)),
         jnp.zeros((_B, _N, _DIN - 3 - _C), jnp.float32)], axis=2)
    table = table.reshape(_B * _N, _DIN)
    gathered = _sc_gather(table, gidx.reshape(_TOT))

    w1p = jnp.concatenate(
        [W1, jnp.zeros((_DIN - 3 - _C, 64), jnp.float32)], axis=0)
    new_features = _mlp(gathered, new_xyz, w1p, b1.reshape(1, 64),
                        W2, b2.reshape(1, 64), W3, b3.reshape(1, 128))
    return new_xyz, new_features


# speedup vs baseline: 3.4094x; 1.0099x over previous
import functools

import jax
import jax.numpy as jnp
from jax import lax
from jax.experimental import pallas as pl
from jax.experimental.pallas import tpu as pltpu
from jax.experimental.pallas import tpu_sc as plsc

_B, _N, _C = 4, 16384, 32
_S, _NS = 1024, 32
_R2 = 0.1 * 0.1
_DIN = 48
_NT = 512
_ST = 256
_BIG = 1e9


_FPP = 4


def _fps_body(xp_ref, yp_ref, zp_ref, new_xyz_ref):
    xs = [xp_ref[a] for a in range(_FPP)]
    ys = [yp_ref[a] for a in range(_FPP)]
    zs = [zp_ref[a] for a in range(_FPP)]
    flat = (lax.broadcasted_iota(jnp.int32, (128, 128), 0) * 128
            + lax.broadcasted_iota(jnp.int32, (128, 128), 1)
            ).astype(jnp.float32)

    def body(i, carry):
        out = []
        for a in range(_FPP):
            dists, cx, cy, cz = carry[a]
            new_xyz_ref[a, pl.ds(i, 1), :] = jnp.concatenate(
                [cx, cy, cz], axis=1)
            dx = xs[a] - cx
            dy = ys[a] - cy
            dz = zs[a] - cz
            d = (dx * dx + dy * dy) + dz * dz
            dists = jnp.minimum(dists, d)
            m = jnp.max(dists)
            cand = jnp.where(dists == m, flat, _BIG)
            m2 = jnp.min(cand)
            onehot = cand == m2
            nx = jnp.sum(jnp.where(onehot, xs[a], 0.0)).reshape(1, 1)
            ny = jnp.sum(jnp.where(onehot, ys[a], 0.0)).reshape(1, 1)
            nz = jnp.sum(jnp.where(onehot, zs[a], 0.0)).reshape(1, 1)
            out.append((dists, nx, ny, nz))
        return tuple(out)

    dists0 = jnp.full((128, 128), 1e10, dtype=jnp.float32)
    init = tuple(
        (dists0, xs[a][0:1, 0:1], ys[a][0:1, 0:1], zs[a][0:1, 0:1])
        for a in range(_FPP))
    lax.fori_loop(0, _S, body, init)


def _fps(xp, yp, zp):
    return pl.pallas_call(
        _fps_body,
        grid=(_B // _FPP,),
        in_specs=[
            pl.BlockSpec((_FPP, 128, 128), lambda b: (b, 0, 0)),
            pl.BlockSpec((_FPP, 128, 128), lambda b: (b, 0, 0)),
            pl.BlockSpec((_FPP, 128, 128), lambda b: (b, 0, 0)),
        ],
        out_specs=pl.BlockSpec((_FPP, _S, 3), lambda b: (b, 0, 0)),
        out_shape=jax.ShapeDtypeStruct((_B, _S, 3), jnp.float32),
        compiler_params=pltpu.CompilerParams(
            dimension_semantics=("arbitrary",),
        ),
    )(xp, yp, zp)


def _bq_body(nx_ref, px_ref, py_ref, pz_ref, tri_ref, gidx_ref):
    b = pl.program_id(0)
    c = nx_ref[0]
    cx = c[:, 0:1]
    cy = c[:, 1:2]
    cz = c[:, 2:3]
    cn = (cx * cx + cy * cy) + cz * cz
    cxb = cx.astype(jnp.bfloat16).astype(jnp.float32)
    cyb = cy.astype(jnp.bfloat16).astype(jnp.float32)
    czb = cz.astype(jnp.bfloat16).astype(jnp.float32)
    tri = tri_ref[...]
    lane = lax.broadcasted_iota(jnp.int32, (32, _NT), 1).astype(jnp.float32)

    def one_tile(t, grp, run):
        sl = pl.ds(t * _NT, _NT)
        pxt = px_ref[0, 0:1, sl]
        pyt = py_ref[0, 0:1, sl]
        pzt = pz_ref[0, 0:1, sl]
        pnt = (pxt * pxt + pyt * pyt) + pzt * pzt
        pxb = pxt.astype(jnp.bfloat16).astype(jnp.float32)
        pyb = pyt.astype(jnp.bfloat16).astype(jnp.float32)
        pzb = pzt.astype(jnp.bfloat16).astype(jnp.float32)
        dot = (cxb * pxb + cyb * pyb) + czb * pzb
        sq = (cn + pnt) - 2.0 * dot
        mask = sq <= _R2
        mf = mask.astype(jnp.bfloat16)
        pref = lax.dot_general(
            mf, tri, (((1,), (0,)), ((), ())),
            preferred_element_type=jnp.float32)
        pref = pref + run
        colf = lane + (t * _NT).astype(jnp.float32)
        slots = []
        for k in range(_NS):
            cand = jnp.where(pref >= (k + 1), colf, _BIG)
            slots.append(jnp.min(cand, axis=1, keepdims=True))
        tile_min = jnp.concatenate(slots, axis=1)
        return jnp.minimum(grp, tile_min), pref[:, _NT - 1:_NT]

    def tile8(t, carry):
        grp, run = carry
        for u in range(8):
            grp, run = one_tile(8 * t + u, grp, run)
        return grp, run

    grp0 = jnp.full((32, _NS), _BIG, dtype=jnp.float32)
    run0 = jnp.zeros((32, 1), dtype=jnp.float32)
    grp, _ = lax.fori_loop(0, _N // _NT // 8, tile8, (grp0, run0))
    first = grp[:, 0:1]
    grp = jnp.where(grp >= _BIG, first, grp)
    gidx_ref[0] = grp.astype(jnp.int32) + b * _N


def _ball_query(new_xyz, px, py, pz, tri):
    return pl.pallas_call(
        _bq_body,
        grid=(_B, _S // 32),
        in_specs=[
            pl.BlockSpec((1, 32, 3), lambda b, s: (b, s, 0)),
            pl.BlockSpec((1, 1, _N), lambda b, s: (b, 0, 0)),
            pl.BlockSpec((1, 1, _N), lambda b, s: (b, 0, 0)),
            pl.BlockSpec((1, 1, _N), lambda b, s: (b, 0, 0)),
            pl.BlockSpec((_NT, _NT), lambda b, s: (0, 0)),
        ],
        out_specs=pl.BlockSpec((1, 32, _NS), lambda b, s: (b, s, 0)),
        out_shape=jax.ShapeDtypeStruct((_B, _S, _NS), jnp.int32),
        compiler_params=pltpu.CompilerParams(
            dimension_semantics=("parallel", "arbitrary"),
        ),
    )(new_xyz, px, py, pz, tri)


_NW = 32
_TOT = _B * _S * _NS
_PERW = _TOT // _NW
_CH = 1024


def _sc_gather(table, idx_flat):
    mesh = plsc.VectorSubcoreMesh(core_axis_name="c", subcore_axis_name="s")

    @functools.partial(
        pl.kernel,
        out_type=jax.ShapeDtypeStruct((_TOT, _DIN), jnp.float32),
        mesh=mesh,
        scratch_types=[
            pltpu.VMEM((_CH,), jnp.int32),
            pltpu.VMEM((_CH, _DIN), jnp.float32),
            pltpu.SemaphoreType.DMA,
        ],
        compiler_params=pltpu.CompilerParams(use_tc_tiling_on_sc=False),
    )
    def k(table_hbm, idx_hbm, out_hbm, idx_v, rows_v, sem):
        wid = lax.axis_index("s") * 2 + lax.axis_index("c")
        base = wid * _PERW

        @pl.loop(0, _PERW, step=_CH)
        def _(off):
            pltpu.sync_copy(idx_hbm.at[pl.ds(base + off, _CH)], idx_v)
            pltpu.async_copy(table_hbm.at[idx_v], rows_v, sem).wait()
            pltpu.sync_copy(rows_v, out_hbm.at[pl.ds(base + off, _CH)])

    return k(table, idx_flat)


def _mlp_body(g_ref, nx_ref, w1_ref, b1_ref, w2_ref, b2_ref, w3_ref, b3_ref,
              out_ref):
    g = g_ref[...]
    c = nx_ref[0]
    cc = jnp.broadcast_to(c.reshape(_ST, 1, 3), (_ST, _NS, 3))
    cc = cc.reshape(_ST * _NS, 3)
    sub = jnp.concatenate(
        [cc, jnp.zeros((_ST * _NS, _DIN - 3), jnp.float32)], axis=1)
    t = g - sub

    def mm(x, w, bias):
        y = lax.dot_general(x, w, (((1,), (0,)), ((), ())),
                            preferred_element_type=jnp.float32)
        return jnp.maximum(y + bias, 0.0)

    h = mm(t, w1_ref[...], b1_ref[...])
    h = mm(h, w2_ref[...], b2_ref[...])
    h = mm(h, w3_ref[...], b3_ref[...])
    pooled = jnp.max(h.reshape(_ST, _NS, 128), axis=1)
    out_ref[0] = pooled.T


def _mlp(gathered, new_xyz, w1p, b1, w2, b2, w3, b3):
    nblk = _S // _ST
    return pl.pallas_call(
        _mlp_body,
        grid=(_B, nblk),
        in_specs=[
            pl.BlockSpec((_ST * _NS, _DIN), lambda b, s: (b * nblk + s, 0)),
            pl.BlockSpec((1, _ST, 3), lambda b, s: (b, s, 0)),
            pl.BlockSpec((_DIN, 64), lambda b, s: (0, 0)),
            pl.BlockSpec((1, 64), lambda b, s: (0, 0)),
            pl.BlockSpec((64, 64), lambda b, s: (0, 0)),
            pl.BlockSpec((1, 64), lambda b, s: (0, 0)),
            pl.BlockSpec((64, 128), lambda b, s: (0, 0)),
            pl.BlockSpec((1, 128), lambda b, s: (0, 0)),
        ],
        out_specs=pl.BlockSpec((1, 128, _ST), lambda b, s: (b, 0, s)),
        out_shape=jax.ShapeDtypeStruct((_B, 128, _S), jnp.float32),
        compiler_params=pltpu.CompilerParams(
            dimension_semantics=("parallel", "arbitrary"),
        ),
    )(gathered, new_xyz, w1p, b1, w2, b2, w3, b3)


def kernel(xyz, features, W1, b1, W2, b2, W3, b3):
    x = xyz[:, :, 0]
    y = xyz[:, :, 1]
    z = xyz[:, :, 2]
    new_xyz = _fps(x.reshape(_B, 128, 128), y.reshape(_B, 128, 128),
                   z.reshape(_B, 128, 128))

    tri = (lax.broadcasted_iota(jnp.int32, (_NT, _NT), 0)
           <= lax.broadcasted_iota(jnp.int32, (_NT, _NT), 1)
           ).astype(jnp.bfloat16)
    gidx = _ball_query(new_xyz, x.reshape(_B, 1, _N), y.reshape(_B, 1, _N),
                       z.reshape(_B, 1, _N), tri)

    table = jnp.concatenate(
        [xyz, jnp.transpose(features, (0, 2, 1)),
         jnp.zeros((_B, _N, _DIN - 3 - _C), jnp.float32)], axis=2)
    table = table.reshape(_B * _N, _DIN)
    gathered = _sc_gather(table, gidx.reshape(_TOT))

    w1p = jnp.concatenate(
        [W1, jnp.zeros((_DIN - 3 - _C, 64), jnp.float32)], axis=0)
    new_features = _mlp(gathered, new_xyz, w1p, b1.reshape(1, 64),
                        W2, b2.reshape(1, 64), W3, b3.reshape(1, 128))
    return new_xyz, new_features


# speedup vs baseline: 3.4453x; 1.0105x over previous
import functools

import jax
import jax.numpy as jnp
from jax import lax
from jax.experimental import pallas as pl
from jax.experimental.pallas import tpu as pltpu
from jax.experimental.pallas import tpu_sc as plsc

_B, _N, _C = 4, 16384, 32
_S, _NS = 1024, 32
_R2 = 0.1 * 0.1
_DIN = 48
_NT = 1024
_ST = 256
_BIG = 1e9


_FPP = 4


def _fps_body(xp_ref, yp_ref, zp_ref, new_xyz_ref):
    xs = [xp_ref[a] for a in range(_FPP)]
    ys = [yp_ref[a] for a in range(_FPP)]
    zs = [zp_ref[a] for a in range(_FPP)]
    flat = (lax.broadcasted_iota(jnp.int32, (128, 128), 0) * 128
            + lax.broadcasted_iota(jnp.int32, (128, 128), 1)
            ).astype(jnp.float32)

    def body(i, carry):
        out = []
        for a in range(_FPP):
            dists, cx, cy, cz = carry[a]
            new_xyz_ref[a, pl.ds(i, 1), :] = jnp.concatenate(
                [cx, cy, cz], axis=1)
            dx = xs[a] - cx
            dy = ys[a] - cy
            dz = zs[a] - cz
            d = (dx * dx + dy * dy) + dz * dz
            dists = jnp.minimum(dists, d)
            m = jnp.max(dists)
            cand = jnp.where(dists == m, flat, _BIG)
            m2 = jnp.min(cand)
            onehot = cand == m2
            nx = jnp.sum(jnp.where(onehot, xs[a], 0.0)).reshape(1, 1)
            ny = jnp.sum(jnp.where(onehot, ys[a], 0.0)).reshape(1, 1)
            nz = jnp.sum(jnp.where(onehot, zs[a], 0.0)).reshape(1, 1)
            out.append((dists, nx, ny, nz))
        return tuple(out)

    dists0 = jnp.full((128, 128), 1e10, dtype=jnp.float32)
    init = tuple(
        (dists0, xs[a][0:1, 0:1], ys[a][0:1, 0:1], zs[a][0:1, 0:1])
        for a in range(_FPP))
    lax.fori_loop(0, _S, body, init)


def _fps(xp, yp, zp):
    return pl.pallas_call(
        _fps_body,
        grid=(_B // _FPP,),
        in_specs=[
            pl.BlockSpec((_FPP, 128, 128), lambda b: (b, 0, 0)),
            pl.BlockSpec((_FPP, 128, 128), lambda b: (b, 0, 0)),
            pl.BlockSpec((_FPP, 128, 128), lambda b: (b, 0, 0)),
        ],
        out_specs=pl.BlockSpec((_FPP, _S, 3), lambda b: (b, 0, 0)),
        out_shape=jax.ShapeDtypeStruct((_B, _S, 3), jnp.float32),
        compiler_params=pltpu.CompilerParams(
            dimension_semantics=("arbitrary",),
        ),
    )(xp, yp, zp)


def _bq_body(nx_ref, px_ref, py_ref, pz_ref, tri_ref, gidx_ref):
    b = pl.program_id(0)
    c = nx_ref[0]
    cx = c[:, 0:1]
    cy = c[:, 1:2]
    cz = c[:, 2:3]
    cn = (cx * cx + cy * cy) + cz * cz
    cxb = cx.astype(jnp.bfloat16).astype(jnp.float32)
    cyb = cy.astype(jnp.bfloat16).astype(jnp.float32)
    czb = cz.astype(jnp.bfloat16).astype(jnp.float32)
    tri = tri_ref[...]
    lane = lax.broadcasted_iota(jnp.int32, (32, _NT), 1).astype(jnp.float32)

    def one_tile(t, grp, run):
        sl = pl.ds(t * _NT, _NT)
        pxt = px_ref[0, 0:1, sl]
        pyt = py_ref[0, 0:1, sl]
        pzt = pz_ref[0, 0:1, sl]
        pnt = (pxt * pxt + pyt * pyt) + pzt * pzt
        pxb = pxt.astype(jnp.bfloat16).astype(jnp.float32)
        pyb = pyt.astype(jnp.bfloat16).astype(jnp.float32)
        pzb = pzt.astype(jnp.bfloat16).astype(jnp.float32)
        dot = (cxb * pxb + cyb * pyb) + czb * pzb
        sq = (cn + pnt) - 2.0 * dot
        mask = sq <= _R2
        mf = mask.astype(jnp.bfloat16)
        pref = lax.dot_general(
            mf, tri, (((1,), (0,)), ((), ())),
            preferred_element_type=jnp.float32)
        pref = pref + run
        colf = lane + (t * _NT).astype(jnp.float32)
        slots = []
        for k in range(_NS):
            cand = jnp.where(pref >= (k + 1), colf, _BIG)
            slots.append(jnp.min(cand, axis=1, keepdims=True))
        tile_min = jnp.concatenate(slots, axis=1)
        return jnp.minimum(grp, tile_min), pref[:, _NT - 1:_NT]

    def tile4(t, carry):
        grp, run = carry
        for u in range(4):
            grp, run = one_tile(4 * t + u, grp, run)
        return grp, run

    grp0 = jnp.full((32, _NS), _BIG, dtype=jnp.float32)
    run0 = jnp.zeros((32, 1), dtype=jnp.float32)
    grp, _ = lax.fori_loop(0, _N // _NT // 4, tile4, (grp0, run0))
    first = grp[:, 0:1]
    grp = jnp.where(grp >= _BIG, first, grp)
    gidx_ref[0] = grp.astype(jnp.int32) + b * _N


def _ball_query(new_xyz, px, py, pz, tri):
    return pl.pallas_call(
        _bq_body,
        grid=(_B, _S // 32),
        in_specs=[
            pl.BlockSpec((1, 32, 3), lambda b, s: (b, s, 0)),
            pl.BlockSpec((1, 1, _N), lambda b, s: (b, 0, 0)),
            pl.BlockSpec((1, 1, _N), lambda b, s: (b, 0, 0)),
            pl.BlockSpec((1, 1, _N), lambda b, s: (b, 0, 0)),
            pl.BlockSpec((_NT, _NT), lambda b, s: (0, 0)),
        ],
        out_specs=pl.BlockSpec((1, 32, _NS), lambda b, s: (b, s, 0)),
        out_shape=jax.ShapeDtypeStruct((_B, _S, _NS), jnp.int32),
        compiler_params=pltpu.CompilerParams(
            dimension_semantics=("parallel", "arbitrary"),
        ),
    )(new_xyz, px, py, pz, tri)


_NW = 32
_TOT = _B * _S * _NS
_PERW = _TOT // _NW
_CH = 1024


def _sc_gather(table, idx_flat):
    mesh = plsc.VectorSubcoreMesh(core_axis_name="c", subcore_axis_name="s")

    @functools.partial(
        pl.kernel,
        out_type=jax.ShapeDtypeStruct((_TOT, _DIN), jnp.float32),
        mesh=mesh,
        scratch_types=[
            pltpu.VMEM((_CH,), jnp.int32),
            pltpu.VMEM((_CH, _DIN), jnp.float32),
            pltpu.SemaphoreType.DMA,
        ],
        compiler_params=pltpu.CompilerParams(use_tc_tiling_on_sc=False),
    )
    def k(table_hbm, idx_hbm, out_hbm, idx_v, rows_v, sem):
        wid = lax.axis_index("s") * 2 + lax.axis_index("c")
        base = wid * _PERW

        @pl.loop(0, _PERW, step=_CH)
        def _(off):
            pltpu.sync_copy(idx_hbm.at[pl.ds(base + off, _CH)], idx_v)
            pltpu.async_copy(table_hbm.at[idx_v], rows_v, sem).wait()
            pltpu.sync_copy(rows_v, out_hbm.at[pl.ds(base + off, _CH)])

    return k(table, idx_flat)


def _mlp_body(g_ref, nx_ref, w1_ref, b1_ref, w2_ref, b2_ref, w3_ref, b3_ref,
              out_ref):
    g = g_ref[...]
    c = nx_ref[0]
    cc = jnp.broadcast_to(c.reshape(_ST, 1, 3), (_ST, _NS, 3))
    cc = cc.reshape(_ST * _NS, 3)
    sub = jnp.concatenate(
        [cc, jnp.zeros((_ST * _NS, _DIN - 3), jnp.float32)], axis=1)
    t = g - sub

    def mm(x, w, bias):
        y = lax.dot_general(x, w, (((1,), (0,)), ((), ())),
                            preferred_element_type=jnp.float32)
        return jnp.maximum(y + bias, 0.0)

    h = mm(t, w1_ref[...], b1_ref[...])
    h = mm(h, w2_ref[...], b2_ref[...])
    h = mm(h, w3_ref[...], b3_ref[...])
    pooled = jnp.max(h.reshape(_ST, _NS, 128), axis=1)
    out_ref[0] = pooled.T


def _mlp(gathered, new_xyz, w1p, b1, w2, b2, w3, b3):
    nblk = _S // _ST
    return pl.pallas_call(
        _mlp_body,
        grid=(_B, nblk),
        in_specs=[
            pl.BlockSpec((_ST * _NS, _DIN), lambda b, s: (b * nblk + s, 0)),
            pl.BlockSpec((1, _ST, 3), lambda b, s: (b, s, 0)),
            pl.BlockSpec((_DIN, 64), lambda b, s: (0, 0)),
            pl.BlockSpec((1, 64), lambda b, s: (0, 0)),
            pl.BlockSpec((64, 64), lambda b, s: (0, 0)),
            pl.BlockSpec((1, 64), lambda b, s: (0, 0)),
            pl.BlockSpec((64, 128), lambda b, s: (0, 0)),
            pl.BlockSpec((1, 128), lambda b, s: (0, 0)),
        ],
        out_specs=pl.BlockSpec((1, 128, _ST), lambda b, s: (b, 0, s)),
        out_shape=jax.ShapeDtypeStruct((_B, 128, _S), jnp.float32),
        compiler_params=pltpu.CompilerParams(
            dimension_semantics=("parallel", "arbitrary"),
        ),
    )(gathered, new_xyz, w1p, b1, w2, b2, w3, b3)


def kernel(xyz, features, W1, b1, W2, b2, W3, b3):
    x = xyz[:, :, 0]
    y = xyz[:, :, 1]
    z = xyz[:, :, 2]
    new_xyz = _fps(x.reshape(_B, 128, 128), y.reshape(_B, 128, 128),
                   z.reshape(_B, 128, 128))

    tri = (lax.broadcasted_iota(jnp.int32, (_NT, _NT), 0)
           <= lax.broadcasted_iota(jnp.int32, (_NT, _NT), 1)
           ).astype(jnp.bfloat16)
    gidx = _ball_query(new_xyz, x.reshape(_B, 1, _N), y.reshape(_B, 1, _N),
                       z.reshape(_B, 1, _N), tri)

    table = jnp.concatenate(
        [xyz, jnp.transpose(features, (0, 2, 1)),
         jnp.zeros((_B, _N, _DIN - 3 - _C), jnp.float32)], axis=2)
    table = table.reshape(_B * _N, _DIN)
    gathered = _sc_gather(table, gidx.reshape(_TOT))

    w1p = jnp.concatenate(
        [W1, jnp.zeros((_DIN - 3 - _C, 64), jnp.float32)], axis=0)
    new_features = _mlp(gathered, new_xyz, w1p, b1.reshape(1, 64),
                        W2, b2.reshape(1, 64), W3, b3.reshape(1, 128))
    return new_xyz, new_features
